# baseline (speedup 1.0000x reference)
import jax
import jax.numpy as jnp
from jax.experimental import pallas as pl
from jax.experimental.pallas import tpu as pltpu

_B, _T, _C, _H = 256, 256, 64, 512
_NCORES = 1
_BB = _B // _NCORES
_TC = 32
_NT = _T // _TC


def _fused_kernel(data_ref, Wgh_ref, bgh_ref, wgx_ref, bgx_ref,
                  Whist_ref, bhist_ref, Wfeat_ref, bfeat_ref,
                  Wcomb_x_ref, Wcomb_m_ref, bcomb_ref,
                  Wih_x_ref, Wih_m_ref, Whh_ref, bg_ref,
                  out_ref, h_s, c_s, d_s, gh_s, gx_s):
    f32 = jnp.float32
    bf16 = jnp.bfloat16

    @pl.when(pl.program_id(1) == 0)
    def _init():
        h_s[...] = jnp.zeros((_BB, _H), f32)
        c_s[...] = jnp.zeros((_BB, _H), f32)
        d_s[...] = jnp.zeros((_BB, _C), f32)

    def gammas(d):
        gamma_h = jnp.exp(-jnp.maximum(
            jnp.dot(d.astype(bf16), Wgh_ref[...], preferred_element_type=f32)
            + bgh_ref[...], 0.0))
        gamma_x = jnp.exp(-jnp.maximum(d * wgx_ref[...] + bgx_ref[...], 0.0))
        return gamma_h, gamma_x

    def body(t, _):
        d = d_s[...]
        gamma_x = gx_s[...]
        h = h_s[...]
        gamma_h = gh_s[...]
        xr = data_ref[t]
        nanm = jnp.isnan(xr)
        m = jnp.where(nanm, 0.0, 1.0).astype(f32)
        x = jnp.where(nanm, 0.0, xr)

        h = h * gamma_h
        hb = h.astype(bf16)
        x_h = jnp.dot(hb, Whist_ref[...], preferred_element_type=f32) \
            + bhist_ref[...]
        x_c = m * x + (1.0 - m) * x_h
        z_h = jnp.dot(x_c.astype(bf16), Wfeat_ref[...],
                      preferred_element_type=f32) \
            + bfeat_ref[...]
        alpha = (jnp.dot(gamma_x.astype(bf16), Wcomb_x_ref[...],
                         preferred_element_type=f32)
                 + jnp.dot(m.astype(bf16), Wcomb_m_ref[...],
                           preferred_element_type=f32)
                 + bcomb_ref[...])
        c_h = alpha * z_h + (1.0 - alpha) * x_h
        c_c = m * x + (1.0 - m) * c_h

        gates = (jnp.dot(c_c.astype(bf16), Wih_x_ref[...],
                         preferred_element_type=f32)
                 + jnp.dot(m.astype(bf16), Wih_m_ref[...],
                           preferred_element_type=f32)
                 + jnp.dot(hb, Whh_ref[...],
                           preferred_element_type=f32)
                 + bg_ref[...])
        gi = gates[:, 0 * _H:1 * _H]
        gf = gates[:, 1 * _H:2 * _H]
        gg = gates[:, 2 * _H:3 * _H]
        go = gates[:, 3 * _H:4 * _H]
        c_new = jax.nn.sigmoid(gf) * c_s[...] \
            + jax.nn.sigmoid(gi) * jnp.tanh(gg)
        c_s[...] = c_new
        h_s[...] = jax.nn.sigmoid(go) * jnp.tanh(c_new)

        out_ref[t] = c_c
        d = jnp.where(m == 1.0, 1.0, d + 1.0)
        gh_next, gx_next = gammas(d)
        d_s[...] = d
        gh_s[...] = gh_next
        gx_s[...] = gx_next
        return ()

    gh0, gx0 = gammas(d_s[...])
    gh_s[...] = gh0
    gx_s[...] = gx0
    jax.lax.fori_loop(0, _TC, body, (), unroll=2)


def kernel(data, W_ih, W_hh, b_ih, b_hh, W_gh, b_gh, W_gx, b_gx,
           W_hist, b_hist, W_feat, b_feat, W_comb, b_comb):
    f32 = jnp.float32
    bf16 = jnp.bfloat16
    data_tm = jnp.moveaxis(data, 1, 0)

    WghT = W_gh.T.astype(bf16)
    wgx_diag = jnp.diagonal(W_gx).reshape(1, _C)
    WhistT = W_hist.T.astype(bf16)
    eye = jnp.eye(_C, dtype=f32)
    WfeatT = (W_feat * (1.0 - eye)).T.astype(bf16)
    WcombT = W_comb.T
    Wcomb_x = WcombT[:_C].astype(bf16)
    Wcomb_m = WcombT[_C:].astype(bf16)
    WihT = W_ih.T
    Wih_x = WihT[:_C].astype(bf16)
    Wih_m = WihT[_C:].astype(bf16)
    WhhT = W_hh.T.astype(bf16)
    b_gates = (b_ih + b_hh).reshape(1, 4 * _H)

    row = lambda v: v.reshape(1, -1).astype(f32)

    full = lambda shape: pl.BlockSpec(shape, lambda i, j: (0,) * len(shape))
    out_tm = pl.pallas_call(
        _fused_kernel,
        grid=(_NCORES, _NT),
        in_specs=[
            pl.BlockSpec((_TC, _BB, _C), lambda i, j: (j, i, 0)),
            full((_C, _H)), full((1, _H)),
            full((1, _C)), full((1, _C)),
            full((_H, _C)), full((1, _C)),
            full((_C, _C)), full((1, _C)),
            full((_C, _C)), full((_C, _C)), full((1, _C)),
            full((_C, 4 * _H)), full((_C, 4 * _H)),
            full((_H, 4 * _H)), full((1, 4 * _H)),
        ],
        out_specs=pl.BlockSpec((_TC, _BB, _C), lambda i, j: (j, i, 0)),
        out_shape=jax.ShapeDtypeStruct((_T, _B, _C), f32),
        scratch_shapes=[
            pltpu.VMEM((_BB, _H), f32),
            pltpu.VMEM((_BB, _H), f32),
            pltpu.VMEM((_BB, _C), f32),
            pltpu.VMEM((_BB, _H), f32),
            pltpu.VMEM((_BB, _C), f32),
        ],
        compiler_params=pltpu.CompilerParams(
            dimension_semantics=("parallel", "arbitrary"),
            vmem_limit_bytes=64 * 1024 * 1024,
        ),
    )(data_tm, WghT, row(b_gh), wgx_diag, row(b_gx),
      WhistT, row(b_hist), WfeatT, row(b_feat),
      Wcomb_x, Wcomb_m, row(b_comb),
      Wih_x, Wih_m, WhhT, b_gates)

    return jnp.moveaxis(out_tm, 0, 1)


# speedup vs baseline: 1.0079x; 1.0079x over previous
import jax
import jax.numpy as jnp
from jax.experimental import pallas as pl
from jax.experimental.pallas import tpu as pltpu

_B, _T, _C, _H = 256, 256, 64, 512
_NCORES = 1
_BB = _B // _NCORES
_TC = 32
_NT = _T // _TC


def _fused_kernel(data_ref, Wgh_ref, bgh_ref, wgx_ref, bgx_ref,
                  Whist_ref, bhist_ref, Wfeat_ref, bfeat_ref,
                  Wcomb_x_ref, Wcomb_m_ref, bcomb_ref,
                  Wih_x_ref, Wih_m_ref, Whh_ref, bg_ref,
                  out_ref, h_s, c_s, d_s, gh_s, gx_s):
    f32 = jnp.float32
    bf16 = jnp.bfloat16

    @pl.when(pl.program_id(1) == 0)
    def _init():
        h_s[...] = jnp.zeros((_BB, _H), f32)
        c_s[...] = jnp.zeros((_BB, _H), f32)
        d_s[...] = jnp.zeros((_BB, _C), f32)

    def gammas(d):
        gamma_h = jnp.exp(-jnp.maximum(
            jnp.dot(d, Wgh_ref[...], preferred_element_type=f32)
            + bgh_ref[...], 0.0))
        gamma_x = jnp.exp(-jnp.maximum(d * wgx_ref[...] + bgx_ref[...], 0.0))
        return gamma_h, gamma_x

    def body(t, _):
        d = d_s[...]
        gamma_x = gx_s[...]
        h = h_s[...]
        gamma_h = gh_s[...]
        xr = data_ref[t]
        nanm = jnp.isnan(xr)
        m = jnp.where(nanm, 0.0, 1.0).astype(f32)
        x = jnp.where(nanm, 0.0, xr)

        h = h * gamma_h
        hb = h.astype(bf16)
        x_h = jnp.dot(hb, Whist_ref[...], preferred_element_type=f32) \
            + bhist_ref[...]
        x_c = m * x + (1.0 - m) * x_h
        z_h = jnp.dot(x_c.astype(bf16), Wfeat_ref[...],
                      preferred_element_type=f32) \
            + bfeat_ref[...]
        alpha = (jnp.dot(gamma_x.astype(bf16), Wcomb_x_ref[...],
                         preferred_element_type=f32)
                 + jnp.dot(m.astype(bf16), Wcomb_m_ref[...],
                           preferred_element_type=f32)
                 + bcomb_ref[...])
        c_h = alpha * z_h + (1.0 - alpha) * x_h
        c_c = m * x + (1.0 - m) * c_h

        gates = (jnp.dot(c_c.astype(bf16), Wih_x_ref[...],
                         preferred_element_type=f32)
                 + jnp.dot(m.astype(bf16), Wih_m_ref[...],
                           preferred_element_type=f32)
                 + jnp.dot(hb, Whh_ref[...],
                           preferred_element_type=f32)
                 + bg_ref[...])
        gi = gates[:, 0 * _H:1 * _H]
        gf = gates[:, 1 * _H:2 * _H]
        gg = gates[:, 2 * _H:3 * _H]
        go = gates[:, 3 * _H:4 * _H]
        c_new = jax.nn.sigmoid(gf) * c_s[...] \
            + jax.nn.sigmoid(gi) * jnp.tanh(gg)
        c_s[...] = c_new
        h_s[...] = jax.nn.sigmoid(go) * jnp.tanh(c_new)

        out_ref[t] = c_c
        d = jnp.where(m == 1.0, 1.0, d + 1.0)
        gh_next, gx_next = gammas(d)
        d_s[...] = d
        gh_s[...] = gh_next
        gx_s[...] = gx_next
        return ()

    gh0, gx0 = gammas(d_s[...])
    gh_s[...] = gh0
    gx_s[...] = gx0
    jax.lax.fori_loop(0, _TC, body, (), unroll=3)


def kernel(data, W_ih, W_hh, b_ih, b_hh, W_gh, b_gh, W_gx, b_gx,
           W_hist, b_hist, W_feat, b_feat, W_comb, b_comb):
    f32 = jnp.float32
    bf16 = jnp.bfloat16
    data_tm = jnp.moveaxis(data, 1, 0)

    WghT = W_gh.T
    wgx_diag = jnp.diagonal(W_gx).reshape(1, _C)
    WhistT = W_hist.T.astype(bf16)
    eye = jnp.eye(_C, dtype=f32)
    WfeatT = (W_feat * (1.0 - eye)).T.astype(bf16)
    WcombT = W_comb.T
    Wcomb_x = WcombT[:_C].astype(bf16)
    Wcomb_m = WcombT[_C:].astype(bf16)
    WihT = W_ih.T
    Wih_x = WihT[:_C].astype(bf16)
    Wih_m = WihT[_C:].astype(bf16)
    WhhT = W_hh.T.astype(bf16)
    b_gates = (b_ih + b_hh).reshape(1, 4 * _H)

    row = lambda v: v.reshape(1, -1).astype(f32)

    full = lambda shape: pl.BlockSpec(shape, lambda i, j: (0,) * len(shape))
    out_tm = pl.pallas_call(
        _fused_kernel,
        grid=(_NCORES, _NT),
        in_specs=[
            pl.BlockSpec((_TC, _BB, _C), lambda i, j: (j, i, 0)),
            full((_C, _H)), full((1, _H)),
            full((1, _C)), full((1, _C)),
            full((_H, _C)), full((1, _C)),
            full((_C, _C)), full((1, _C)),
            full((_C, _C)), full((_C, _C)), full((1, _C)),
            full((_C, 4 * _H)), full((_C, 4 * _H)),
            full((_H, 4 * _H)), full((1, 4 * _H)),
        ],
        out_specs=pl.BlockSpec((_TC, _BB, _C), lambda i, j: (j, i, 0)),
        out_shape=jax.ShapeDtypeStruct((_T, _B, _C), f32),
        scratch_shapes=[
            pltpu.VMEM((_BB, _H), f32),
            pltpu.VMEM((_BB, _H), f32),
            pltpu.VMEM((_BB, _C), f32),
            pltpu.VMEM((_BB, _H), f32),
            pltpu.VMEM((_BB, _C), f32),
        ],
        compiler_params=pltpu.CompilerParams(
            dimension_semantics=("parallel", "arbitrary"),
            vmem_limit_bytes=64 * 1024 * 1024,
        ),
    )(data_tm, WghT, row(b_gh), wgx_diag, row(b_gx),
      WhistT, row(b_hist), WfeatT, row(b_feat),
      Wcomb_x, Wcomb_m, row(b_comb),
      Wih_x, Wih_m, WhhT, b_gates)

    return jnp.moveaxis(out_tm, 0, 1)


# speedup vs baseline: 1.0247x; 1.0166x over previous
import jax
import jax.numpy as jnp
from jax.experimental import pallas as pl
from jax.experimental.pallas import tpu as pltpu

_B, _T, _C, _H = 256, 256, 64, 512
_NCORES = 1
_BB = _B // _NCORES
_TC = 32
_NT = _T // _TC


def _fused_kernel(data_ref, Wgh_ref, bgh_ref, wgx_ref, bgx_ref,
                  Whist_ref, bhist_ref, Wfeat_ref, bfeat_ref,
                  Wcomb_x_ref, Wcomb_m_ref, bcomb_ref,
                  Wih_x_ref, Wih_m_ref, Whh_ref, bg_ref,
                  out_ref, h_s, c_s, d_s, gh_s, gx_s):
    f32 = jnp.float32
    bf16 = jnp.bfloat16

    @pl.when(pl.program_id(1) == 0)
    def _init():
        h_s[...] = jnp.zeros((_BB, _H), f32)
        c_s[...] = jnp.zeros((_BB, _H), f32)
        d_s[...] = jnp.zeros((_BB, _C), f32)

    def gammas(d):
        gamma_h = jnp.exp(-jnp.maximum(
            jnp.dot(d, Wgh_ref[...], preferred_element_type=f32)
            + bgh_ref[...], 0.0))
        gamma_x = jnp.exp(-jnp.maximum(d * wgx_ref[...] + bgx_ref[...], 0.0))
        return gamma_h, gamma_x

    def body(t, _):
        d = d_s[...]
        gamma_x = gx_s[...]
        h = h_s[...]
        gamma_h = gh_s[...]
        xr = data_ref[t]
        nanm = jnp.isnan(xr)
        m = jnp.where(nanm, 0.0, 1.0).astype(f32)
        x = jnp.where(nanm, 0.0, xr)

        h = h * gamma_h
        hb = h.astype(bf16)
        x_h = jnp.dot(hb, Whist_ref[...], preferred_element_type=f32) \
            + bhist_ref[...]
        x_c = m * x + (1.0 - m) * x_h
        z_h = jnp.dot(x_c.astype(bf16), Wfeat_ref[...],
                      preferred_element_type=f32) \
            + bfeat_ref[...]
        alpha = (jnp.dot(gamma_x.astype(bf16), Wcomb_x_ref[...],
                         preferred_element_type=f32)
                 + jnp.dot(m.astype(bf16), Wcomb_m_ref[...],
                           preferred_element_type=f32)
                 + bcomb_ref[...])
        c_h = alpha * z_h + (1.0 - alpha) * x_h
        c_c = m * x + (1.0 - m) * c_h

        gates = (jnp.dot(c_c.astype(bf16), Wih_x_ref[...],
                         preferred_element_type=f32)
                 + jnp.dot(m.astype(bf16), Wih_m_ref[...],
                           preferred_element_type=f32)
                 + jnp.dot(hb, Whh_ref[...],
                           preferred_element_type=f32)
                 + bg_ref[...])
        gi = gates[:, 0 * _H:1 * _H]
        gf = gates[:, 1 * _H:2 * _H]
        gg = gates[:, 2 * _H:3 * _H]
        go = gates[:, 3 * _H:4 * _H]
        c_new = jax.nn.sigmoid(gf) * c_s[...] \
            + jax.nn.sigmoid(gi) * jnp.tanh(gg)
        c_s[...] = c_new
        h_s[...] = jax.nn.sigmoid(go) * jnp.tanh(c_new)

        out_ref[t] = c_c
        d = jnp.where(m == 1.0, 1.0, d + 1.0)
        gh_next, gx_next = gammas(d)
        d_s[...] = d
        gh_s[...] = gh_next
        gx_s[...] = gx_next
        return ()

    gh0, gx0 = gammas(d_s[...])
    gh_s[...] = gh0
    gx_s[...] = gx0
    jax.lax.fori_loop(0, _TC, body, (), unroll=4)


def kernel(data, W_ih, W_hh, b_ih, b_hh, W_gh, b_gh, W_gx, b_gx,
           W_hist, b_hist, W_feat, b_feat, W_comb, b_comb):
    f32 = jnp.float32
    bf16 = jnp.bfloat16
    data_tm = jnp.moveaxis(data, 1, 0)

    WghT = W_gh.T
    wgx_diag = jnp.diagonal(W_gx).reshape(1, _C)
    WhistT = W_hist.T.astype(bf16)
    eye = jnp.eye(_C, dtype=f32)
    WfeatT = (W_feat * (1.0 - eye)).T.astype(bf16)
    WcombT = W_comb.T
    Wcomb_x = WcombT[:_C].astype(bf16)
    Wcomb_m = WcombT[_C:].astype(bf16)
    WihT = W_ih.T
    Wih_x = WihT[:_C].astype(bf16)
    Wih_m = WihT[_C:].astype(bf16)
    WhhT = W_hh.T.astype(bf16)
    b_gates = (b_ih + b_hh).reshape(1, 4 * _H)

    row = lambda v: v.reshape(1, -1).astype(f32)

    full = lambda shape: pl.BlockSpec(shape, lambda i, j: (0,) * len(shape))
    out_tm = pl.pallas_call(
        _fused_kernel,
        grid=(_NCORES, _NT),
        in_specs=[
            pl.BlockSpec((_TC, _BB, _C), lambda i, j: (j, i, 0)),
            full((_C, _H)), full((1, _H)),
            full((1, _C)), full((1, _C)),
            full((_H, _C)), full((1, _C)),
            full((_C, _C)), full((1, _C)),
            full((_C, _C)), full((_C, _C)), full((1, _C)),
            full((_C, 4 * _H)), full((_C, 4 * _H)),
            full((_H, 4 * _H)), full((1, 4 * _H)),
        ],
        out_specs=pl.BlockSpec((_TC, _BB, _C), lambda i, j: (j, i, 0)),
        out_shape=jax.ShapeDtypeStruct((_T, _B, _C), f32),
        scratch_shapes=[
            pltpu.VMEM((_BB, _H), f32),
            pltpu.VMEM((_BB, _H), f32),
            pltpu.VMEM((_BB, _C), f32),
            pltpu.VMEM((_BB, _H), f32),
            pltpu.VMEM((_BB, _C), f32),
        ],
        compiler_params=pltpu.CompilerParams(
            dimension_semantics=("parallel", "arbitrary"),
            vmem_limit_bytes=64 * 1024 * 1024,
        ),
    )(data_tm, WghT, row(b_gh), wgx_diag, row(b_gx),
      WhistT, row(b_hist), WfeatT, row(b_feat),
      Wcomb_x, Wcomb_m, row(b_comb),
      Wih_x, Wih_m, WhhT, b_gates)

    return jnp.moveaxis(out_tm, 0, 1)


# speedup vs baseline: 1.0314x; 1.0066x over previous
import jax
import jax.numpy as jnp
from jax.experimental import pallas as pl
from jax.experimental.pallas import tpu as pltpu

_B, _T, _C, _H = 256, 256, 64, 512
_NCORES = 1
_BB = _B // _NCORES
_TC = 32
_NT = _T // _TC


def _fused_kernel(data_ref, Wgh_ref, bgh_ref, wgx_ref, bgx_ref,
                  Whist_ref, bhist_ref, Wfeat_ref, bfeat_ref,
                  Wcomb_x_ref, Wcomb_m_ref, bcomb_ref,
                  Wih_x_ref, Wih_m_ref, Whh_ref, bg_ref,
                  out_ref, h_s, c_s, d_s, gh_s, gx_s):
    f32 = jnp.float32
    bf16 = jnp.bfloat16

    @pl.when(pl.program_id(1) == 0)
    def _init():
        h_s[...] = jnp.zeros((_BB, _H), f32)
        c_s[...] = jnp.zeros((_BB, _H), f32)
        d_s[...] = jnp.zeros((_BB, _C), f32)

    def gammas(d):
        gamma_h = jnp.exp(-jnp.maximum(
            jnp.dot(d, Wgh_ref[...], preferred_element_type=f32)
            + bgh_ref[...], 0.0))
        gamma_x = jnp.exp(-jnp.maximum(d * wgx_ref[...] + bgx_ref[...], 0.0))
        return gamma_h, gamma_x

    def body(t, _):
        d = d_s[...]
        gamma_x = gx_s[...]
        h = h_s[...]
        gamma_h = gh_s[...]
        xr = data_ref[t]
        nanm = jnp.isnan(xr)
        m = jnp.where(nanm, 0.0, 1.0).astype(f32)
        x = jnp.where(nanm, 0.0, xr)

        h = h * gamma_h
        hb = h.astype(bf16)
        x_h = jnp.dot(hb, Whist_ref[...], preferred_element_type=f32) \
            + bhist_ref[...]
        x_c = m * x + (1.0 - m) * x_h
        z_h = jnp.dot(x_c.astype(bf16), Wfeat_ref[...],
                      preferred_element_type=f32) \
            + bfeat_ref[...]
        alpha = (jnp.dot(gamma_x.astype(bf16), Wcomb_x_ref[...],
                         preferred_element_type=f32)
                 + jnp.dot(m.astype(bf16), Wcomb_m_ref[...],
                           preferred_element_type=f32)
                 + bcomb_ref[...])
        c_h = alpha * z_h + (1.0 - alpha) * x_h
        c_c = m * x + (1.0 - m) * c_h

        gates = (jnp.dot(c_c.astype(bf16), Wih_x_ref[...],
                         preferred_element_type=f32)
                 + jnp.dot(m.astype(bf16), Wih_m_ref[...],
                           preferred_element_type=f32)
                 + jnp.dot(hb, Whh_ref[...],
                           preferred_element_type=f32)
                 + bg_ref[...])
        gi = gates[:, 0 * _H:1 * _H]
        gf = gates[:, 1 * _H:2 * _H]
        gg = gates[:, 2 * _H:3 * _H]
        go = gates[:, 3 * _H:4 * _H]
        c_new = jax.nn.sigmoid(gf) * c_s[...] \
            + jax.nn.sigmoid(gi) * jnp.tanh(gg)
        c_s[...] = c_new
        h_s[...] = jax.nn.sigmoid(go) * jnp.tanh(c_new)

        out_ref[t] = c_c
        d = jnp.where(m == 1.0, 1.0, d + 1.0)
        gh_next, gx_next = gammas(d)
        d_s[...] = d
        gh_s[...] = gh_next
        gx_s[...] = gx_next
        return ()

    gh0, gx0 = gammas(d_s[...])
    gh_s[...] = gh0
    gx_s[...] = gx0
    jax.lax.fori_loop(0, _TC, body, (), unroll=8)


def kernel(data, W_ih, W_hh, b_ih, b_hh, W_gh, b_gh, W_gx, b_gx,
           W_hist, b_hist, W_feat, b_feat, W_comb, b_comb):
    f32 = jnp.float32
    bf16 = jnp.bfloat16
    data_tm = jnp.moveaxis(data, 1, 0)

    WghT = W_gh.T
    wgx_diag = jnp.diagonal(W_gx).reshape(1, _C)
    WhistT = W_hist.T.astype(bf16)
    eye = jnp.eye(_C, dtype=f32)
    WfeatT = (W_feat * (1.0 - eye)).T.astype(bf16)
    WcombT = W_comb.T
    Wcomb_x = WcombT[:_C].astype(bf16)
    Wcomb_m = WcombT[_C:].astype(bf16)
    WihT = W_ih.T
    Wih_x = WihT[:_C].astype(bf16)
    Wih_m = WihT[_C:].astype(bf16)
    WhhT = W_hh.T.astype(bf16)
    b_gates = (b_ih + b_hh).reshape(1, 4 * _H)

    row = lambda v: v.reshape(1, -1).astype(f32)

    full = lambda shape: pl.BlockSpec(shape, lambda i, j: (0,) * len(shape))
    out_tm = pl.pallas_call(
        _fused_kernel,
        grid=(_NCORES, _NT),
        in_specs=[
            pl.BlockSpec((_TC, _BB, _C), lambda i, j: (j, i, 0)),
            full((_C, _H)), full((1, _H)),
            full((1, _C)), full((1, _C)),
            full((_H, _C)), full((1, _C)),
            full((_C, _C)), full((1, _C)),
            full((_C, _C)), full((_C, _C)), full((1, _C)),
            full((_C, 4 * _H)), full((_C, 4 * _H)),
            full((_H, 4 * _H)), full((1, 4 * _H)),
        ],
        out_specs=pl.BlockSpec((_TC, _BB, _C), lambda i, j: (j, i, 0)),
        out_shape=jax.ShapeDtypeStruct((_T, _B, _C), f32),
        scratch_shapes=[
            pltpu.VMEM((_BB, _H), f32),
            pltpu.VMEM((_BB, _H), f32),
            pltpu.VMEM((_BB, _C), f32),
            pltpu.VMEM((_BB, _H), f32),
            pltpu.VMEM((_BB, _C), f32),
        ],
        compiler_params=pltpu.CompilerParams(
            dimension_semantics=("parallel", "arbitrary"),
            vmem_limit_bytes=64 * 1024 * 1024,
        ),
    )(data_tm, WghT, row(b_gh), wgx_diag, row(b_gx),
      WhistT, row(b_hist), WfeatT, row(b_feat),
      Wcomb_x, Wcomb_m, row(b_comb),
      Wih_x, Wih_m, WhhT, b_gates)

    return jnp.moveaxis(out_tm, 0, 1)


# speedup vs baseline: 1.0391x; 1.0074x over previous
import jax
import jax.numpy as jnp
from jax.experimental import pallas as pl
from jax.experimental.pallas import tpu as pltpu

_B, _T, _C, _H = 256, 256, 64, 512
_NCORES = 1
_BB = _B // _NCORES
_TC = 32
_NT = _T // _TC


def _fused_kernel(data_ref, Wgh_ref, bgh_ref, wgx_ref, bgx_ref,
                  Whist_ref, bhist_ref, Wfeat_ref, bfeat_ref,
                  Wcomb_x_ref, Wcomb_m_ref, bcomb_ref,
                  Wih_x_ref, Wih_m_ref, Whh_ref, bg_ref,
                  out_ref, h_s, c_s, d_s, gh_s, gx_s):
    f32 = jnp.float32
    bf16 = jnp.bfloat16

    @pl.when(pl.program_id(1) == 0)
    def _init():
        h_s[...] = jnp.zeros((_BB, _H), f32)
        c_s[...] = jnp.zeros((_BB, _H), f32)
        d_s[...] = jnp.zeros((_BB, _C), f32)

    def gammas(d):
        gamma_h = jnp.exp(-jnp.maximum(
            jnp.dot(d, Wgh_ref[...], preferred_element_type=f32)
            + bgh_ref[...], 0.0))
        gamma_x = jnp.exp(-jnp.maximum(d * wgx_ref[...] + bgx_ref[...], 0.0))
        return gamma_h, gamma_x

    def body(t, _):
        d = d_s[...]
        gamma_x = gx_s[...]
        h = h_s[...]
        gamma_h = gh_s[...]
        xr = data_ref[t]
        nanm = jnp.isnan(xr)
        m = jnp.where(nanm, 0.0, 1.0).astype(f32)
        x = jnp.where(nanm, 0.0, xr)

        h = h * gamma_h
        hb = h.astype(bf16)
        x_h = jnp.dot(hb, Whist_ref[...], preferred_element_type=f32) \
            + bhist_ref[...]
        x_c = m * x + (1.0 - m) * x_h
        z_h = jnp.dot(x_c.astype(bf16), Wfeat_ref[...],
                      preferred_element_type=f32) \
            + bfeat_ref[...]
        alpha = (jnp.dot(gamma_x.astype(bf16), Wcomb_x_ref[...],
                         preferred_element_type=f32)
                 + jnp.dot(m.astype(bf16), Wcomb_m_ref[...],
                           preferred_element_type=f32)
                 + bcomb_ref[...])
        c_h = alpha * z_h + (1.0 - alpha) * x_h
        c_c = m * x + (1.0 - m) * c_h

        gates = (jnp.dot(c_c.astype(bf16), Wih_x_ref[...],
                         preferred_element_type=f32)
                 + jnp.dot(m.astype(bf16), Wih_m_ref[...],
                           preferred_element_type=f32)
                 + jnp.dot(hb, Whh_ref[...],
                           preferred_element_type=f32)
                 + bg_ref[...])
        gi = gates[:, 0 * _H:1 * _H]
        gf = gates[:, 1 * _H:2 * _H]
        gg = gates[:, 2 * _H:3 * _H]
        go = gates[:, 3 * _H:4 * _H]
        c_new = jax.nn.sigmoid(gf) * c_s[...] \
            + jax.nn.sigmoid(gi) * jnp.tanh(gg)
        c_s[...] = c_new
        h_s[...] = jax.nn.sigmoid(go) * jnp.tanh(c_new)

        out_ref[t] = c_c
        d = jnp.where(m == 1.0, 1.0, d + 1.0)
        gh_next, gx_next = gammas(d)
        d_s[...] = d
        gh_s[...] = gh_next
        gx_s[...] = gx_next
        return ()

    gh0, gx0 = gammas(d_s[...])
    gh_s[...] = gh0
    gx_s[...] = gx0
    jax.lax.fori_loop(0, _TC, body, (), unroll=16)


def kernel(data, W_ih, W_hh, b_ih, b_hh, W_gh, b_gh, W_gx, b_gx,
           W_hist, b_hist, W_feat, b_feat, W_comb, b_comb):
    f32 = jnp.float32
    bf16 = jnp.bfloat16
    data_tm = jnp.moveaxis(data, 1, 0)

    WghT = W_gh.T
    wgx_diag = jnp.diagonal(W_gx).reshape(1, _C)
    WhistT = W_hist.T.astype(bf16)
    eye = jnp.eye(_C, dtype=f32)
    WfeatT = (W_feat * (1.0 - eye)).T.astype(bf16)
    WcombT = W_comb.T
    Wcomb_x = WcombT[:_C].astype(bf16)
    Wcomb_m = WcombT[_C:].astype(bf16)
    WihT = W_ih.T
    Wih_x = WihT[:_C].astype(bf16)
    Wih_m = WihT[_C:].astype(bf16)
    WhhT = W_hh.T.astype(bf16)
    b_gates = (b_ih + b_hh).reshape(1, 4 * _H)

    row = lambda v: v.reshape(1, -1).astype(f32)

    full = lambda shape: pl.BlockSpec(shape, lambda i, j: (0,) * len(shape))
    out_tm = pl.pallas_call(
        _fused_kernel,
        grid=(_NCORES, _NT),
        in_specs=[
            pl.BlockSpec((_TC, _BB, _C), lambda i, j: (j, i, 0)),
            full((_C, _H)), full((1, _H)),
            full((1, _C)), full((1, _C)),
            full((_H, _C)), full((1, _C)),
            full((_C, _C)), full((1, _C)),
            full((_C, _C)), full((_C, _C)), full((1, _C)),
            full((_C, 4 * _H)), full((_C, 4 * _H)),
            full((_H, 4 * _H)), full((1, 4 * _H)),
        ],
        out_specs=pl.BlockSpec((_TC, _BB, _C), lambda i, j: (j, i, 0)),
        out_shape=jax.ShapeDtypeStruct((_T, _B, _C), f32),
        scratch_shapes=[
            pltpu.VMEM((_BB, _H), f32),
            pltpu.VMEM((_BB, _H), f32),
            pltpu.VMEM((_BB, _C), f32),
            pltpu.VMEM((_BB, _H), f32),
            pltpu.VMEM((_BB, _C), f32),
        ],
        compiler_params=pltpu.CompilerParams(
            dimension_semantics=("parallel", "arbitrary"),
            vmem_limit_bytes=64 * 1024 * 1024,
        ),
    )(data_tm, WghT, row(b_gh), wgx_diag, row(b_gx),
      WhistT, row(b_hist), WfeatT, row(b_feat),
      Wcomb_x, Wcomb_m, row(b_comb),
      Wih_x, Wih_m, WhhT, b_gates)

    return jnp.moveaxis(out_tm, 0, 1)
